# Initial kernel scaffold; baseline (speedup 1.0000x reference)
import jax
import jax.numpy as jnp
from jax.experimental import pallas as pl


def kernel(inp, divisor, w0_gate, w0, w1, w2_gate, w2):
    raise NotImplementedError("write your pallas kernel here")


# speedup vs baseline: 1.4275x; 1.4275x over previous
import functools

import jax
import jax.numpy as jnp
from jax import lax
from jax.experimental import pallas as pl
from jax.experimental.pallas import tpu as pltpu

F = 768
S = 2048
E = 8
K = 7
G = 3
TS = 256
NT = S // TS

_HI = lax.Precision.HIGHEST


def _dot(a, b, prec=_HI):
    return lax.dot_general(a, b, (((1,), (0,)), ((), ())),
                           preferred_element_type=jnp.float32, precision=prec)


def _bdot(a, b):
    return lax.dot_general(a.astype(jnp.bfloat16), b.astype(jnp.bfloat16),
                           (((1,), (0,)), ((), ())),
                           preferred_element_type=jnp.float32)


def _router(x, gate):
    logits = lax.dot_general(x.astype(jnp.bfloat16),
                             gate.astype(jnp.bfloat16),
                             (((1,), (1,)), ((), ())),
                             preferred_element_type=jnp.float32)
    m = jnp.max(logits, axis=1, keepdims=True)
    lane = lax.broadcasted_iota(jnp.int32, logits.shape, 1)
    assign = jnp.min(jnp.where(logits == m, lane, E), axis=1, keepdims=True)
    return logits, assign


def _moe1_body(x_ref, gate_ref, w_ref, y_ref, lg_ref):
    x = x_ref[...]
    logits, assign = _router(x, gate_ref[...])
    lg_ref[...] = logits
    acc = jnp.zeros((TS, F), jnp.float32)
    for e in range(E):
        pe = _bdot(x, w_ref[e])
        acc = acc + jnp.where(assign == e, pe, 0.0)
    y_ref[...] = jnp.maximum(acc, 0.0)


def _moe1_call(x, gate, w):
    return pl.pallas_call(
        _moe1_body,
        grid=(NT,),
        in_specs=[
            pl.BlockSpec((TS, F), lambda t: (t, 0)),
            pl.BlockSpec((E, F), lambda t: (0, 0)),
            pl.BlockSpec((E, F, F), lambda t: (0, 0, 0)),
        ],
        out_specs=[
            pl.BlockSpec((TS, F), lambda t: (t, 0)),
            pl.BlockSpec((TS, E), lambda t: (t, 0)),
        ],
        out_shape=[
            jax.ShapeDtypeStruct((S, F), jnp.float32),
            jax.ShapeDtypeStruct((S, E), jnp.float32),
        ],
    )(x, gate, w)


def _conv_body(x_ref, w_ref, gate_ref, xb_ref, lg_ref, oh_ref):
    t = pl.program_id(0)
    win = x_ref[pl.ds(t * TS, TS + 8), :]
    acc = jnp.zeros((TS, F), jnp.float32)
    for k in range(K):
        acc = acc + _bdot(lax.slice_in_dim(win, k, k + TS, axis=0), w_ref[k])
    x3 = jnp.maximum(acc, 0.0)
    logits, assign = _router(x3, gate_ref[...])
    lg_ref[...] = logits
    lane = lax.broadcasted_iota(jnp.int32, logits.shape, 1)
    oh_ref[...] = (assign == lane).astype(jnp.bfloat16)
    xb_ref[...] = x3.astype(jnp.bfloat16)


def _conv_call(x, wk, gate):
    xp = jnp.pad(x, ((K - 1, 2), (0, 0)))
    return pl.pallas_call(
        _conv_body,
        grid=(NT,),
        in_specs=[pl.BlockSpec((S + K + 1, F), lambda t: (0, 0)),
                  pl.BlockSpec((K, F, F), lambda t: (0, 0, 0)),
                  pl.BlockSpec((E, F), lambda t: (0, 0))],
        out_specs=[pl.BlockSpec((TS, F), lambda t: (t, 0)),
                   pl.BlockSpec((TS, E), lambda t: (t, 0)),
                   pl.BlockSpec((TS, E), lambda t: (t, 0))],
        out_shape=[jax.ShapeDtypeStruct((S, F), jnp.bfloat16),
                   jax.ShapeDtypeStruct((S, E), jnp.float32),
                   jax.ShapeDtypeStruct((S, E), jnp.bfloat16)],
    )(xp, wk, gate)


def _pos_body(oh_ref, pos_ref, offs_ref):
    t = pl.program_id(0)
    ohf = oh_ref[...]
    ones_row = jnp.ones((1, S), jnp.bfloat16)
    counts = _bdot(ones_row, ohf)
    sl = (lax.broadcasted_iota(jnp.int32, (E, E), 0)
          < lax.broadcasted_iota(jnp.int32, (E, E), 1)).astype(jnp.float32)
    offs_row = _dot(counts, sl)
    offs_ref[...] = offs_row
    oh_tile = oh_ref[pl.ds(t * TS, TS), :].astype(jnp.float32)
    col = lax.broadcasted_iota(jnp.int32, (TS, S), 1)
    row = lax.broadcasted_iota(jnp.int32, (TS, S), 0) + t * TS
    lrow = (col < row).astype(jnp.bfloat16)
    ranks_e = _bdot(lrow, ohf)
    rank = jnp.sum(ranks_e * oh_tile, axis=1, keepdims=True)
    cnt_lt = jnp.sum(offs_row * oh_tile, axis=1, keepdims=True)
    pos_ref[...] = rank + cnt_lt


def _pos_call(oh):
    pos, offs = pl.pallas_call(
        _pos_body,
        grid=(NT,),
        in_specs=[pl.BlockSpec((S, E), lambda t: (0, 0))],
        out_specs=[pl.BlockSpec((TS, 1), lambda t: (t, 0)),
                   pl.BlockSpec((1, E), lambda t: (0, 0))],
        out_shape=[jax.ShapeDtypeStruct((S, 1), jnp.float32),
                   jax.ShapeDtypeStruct((1, E), jnp.float32)],
    )(oh)
    return pos, offs


def _gmm_body(offs_ref, pos_ref, x_ref, w_ref, y_ref):
    t = pl.program_id(1)
    rowi = (lax.broadcasted_iota(jnp.int32, (TS, S), 0) + t * TS)
    m = (pos_ref[...] == rowi.astype(jnp.float32)).astype(jnp.bfloat16)
    xs = _bdot(m, x_ref[...]).astype(jnp.bfloat16)
    pcol = lax.broadcasted_iota(jnp.int32, (TS, 1), 0) + t * TS
    y_ref[...] = jnp.zeros((TS, F), jnp.float32)
    for e in range(E):
        st = offs_ref[e]
        en = offs_ref[e + 1]

        @pl.when((en > t * TS) & (st < (t + 1) * TS))
        def _():
            pe = _bdot(xs, w_ref[e])
            mask = (pcol >= st) & (pcol < en)
            y_ref[...] += jnp.where(mask, pe, 0.0)


def _gmm_call(offs9, pos_row, xb, w):
    return pl.pallas_call(
        _gmm_body,
        grid_spec=pltpu.PrefetchScalarGridSpec(
            num_scalar_prefetch=1,
            grid=(G, NT),
            in_specs=[
                pl.BlockSpec((1, S), lambda j, t, offs: (0, 0)),
                pl.BlockSpec((S, F), lambda j, t, offs: (0, 0)),
                pl.BlockSpec((E, F, F), lambda j, t, offs: (0, 0, j)),
            ],
            out_specs=pl.BlockSpec((TS, F), lambda j, t, offs: (t, j)),
        ),
        out_shape=jax.ShapeDtypeStruct((S, G * F), jnp.float32),
    )(offs9, pos_row, xb, w)


def _norm_body(y_ref, u_ref, div_ref, out_ref):
    depth = y_ref[:, 0:F]
    scale = y_ref[:, F:2 * F]
    shift = y_ref[:, 2 * F:3 * F]
    cum = _dot(depth, u_ref[...])
    t = cum / div_ref[...] * scale + shift
    mu = jnp.mean(t, axis=1, keepdims=True)
    c = t - mu
    nrm = jnp.sqrt(jnp.sum(c * c, axis=1, keepdims=True))
    out_ref[...] = (c / (nrm * (F ** -0.5) + 1e-5)).astype(jnp.bfloat16)


def _norm_call(y, u, div_row):
    return pl.pallas_call(
        _norm_body,
        grid=(NT,),
        in_specs=[pl.BlockSpec((TS, G * F), lambda t: (t, 0)),
                  pl.BlockSpec((F, F), lambda t: (0, 0)),
                  pl.BlockSpec((1, F), lambda t: (0, 0))],
        out_specs=pl.BlockSpec((TS, F), lambda t: (t, 0)),
        out_shape=jax.ShapeDtypeStruct((S, F), jnp.bfloat16),
    )(y, u, div_row)


def _unsort_body(pos_ref, ys_ref, out_ref):
    pos = pos_ref[...]
    col = lax.broadcasted_iota(jnp.int32, (TS, S), 1).astype(jnp.float32)
    m2 = (pos == col).astype(jnp.bfloat16)
    out_ref[...] = _bdot(m2, ys_ref[...])


def _unsort_call(pos, ys):
    return pl.pallas_call(
        _unsort_body,
        grid=(NT,),
        in_specs=[pl.BlockSpec((TS, 1), lambda t: (t, 0)),
                  pl.BlockSpec((S, F), lambda t: (0, 0))],
        out_specs=pl.BlockSpec((TS, F), lambda t: (t, 0)),
        out_shape=jax.ShapeDtypeStruct((S, F), jnp.float32),
    )(pos, ys)


def _loss_body(lg0_ref, lg1_ref, l0_ref, l1_ref):
    for lg_ref, l_ref in ((lg0_ref, l0_ref), (lg1_ref, l1_ref)):
        logits = lg_ref[...]
        m = jnp.max(logits, axis=1, keepdims=True)
        ex = jnp.exp(logits - m)
        p = ex / jnp.sum(ex, axis=1, keepdims=True)
        lane = lax.broadcasted_iota(jnp.int32, logits.shape, 1)
        assign = jnp.min(jnp.where(logits == m, lane, E), axis=1,
                         keepdims=True)
        oh = (assign == lane).astype(jnp.float32)
        gsum = jnp.sum(p, axis=0, keepdims=True)
        csum = jnp.sum(oh, axis=0, keepdims=True)
        l_ref[...] = jnp.sum(gsum * csum, axis=1, keepdims=True) / (S * S)


def _loss_call(lg0, lg1):
    l0, l1 = pl.pallas_call(
        _loss_body,
        in_specs=[pl.BlockSpec((S, E), lambda: (0, 0)),
                  pl.BlockSpec((S, E), lambda: (0, 0))],
        out_specs=[pl.BlockSpec((1, 1), lambda: (0, 0)),
                   pl.BlockSpec((1, 1), lambda: (0, 0))],
        out_shape=[jax.ShapeDtypeStruct((1, 1), jnp.float32),
                   jax.ShapeDtypeStruct((1, 1), jnp.float32)],
    )(lg0, lg1)
    return l0[0, 0], l1[0, 0]


def kernel(inp, divisor, w0_gate, w0, w1, w2_gate, w2):
    x = inp[0].T
    gate0 = w0_gate[:, :, 0]
    gate1 = w2_gate[:, :, 0]
    w1k = jnp.transpose(w1, (2, 1, 0))
    w2b = w2.astype(jnp.bfloat16)
    div_row = divisor[0].T
    u = (lax.broadcasted_iota(jnp.int32, (F, F), 0)
         <= lax.broadcasted_iota(jnp.int32, (F, F), 1)).astype(jnp.float32)

    y0, lg0 = _moe1_call(x, gate0, w0)
    xb, lg1, oh1 = _conv_call(y0, w1k, gate1)
    pos, offs = _pos_call(oh1)
    offs9 = jnp.concatenate(
        [offs[0].astype(jnp.int32),
         jnp.full((8,), S, jnp.int32)])
    y2s = _gmm_call(offs9, pos.reshape(1, S), xb, w2b)
    outs = _norm_call(y2s, u, div_row)
    out = _unsort_call(pos, outs)
    l0, l1 = _loss_call(lg0, lg1)
    return (l0, l1, out[None].transpose(0, 2, 1))
